# baseline (speedup 1.0000x reference)
import functools

import jax
import jax.numpy as jnp
from jax import lax
from jax.experimental import pallas as pl
from jax.experimental.pallas import tpu as pltpu
from jax.experimental.pallas import tpu_sc as plsc

_VOCAB = 1000000
_EMBED_DIM = 32
_BATCH = 16384
_HIST = 50

_info = plsc.get_sparse_core_info()
_NC, _NS_SUB = _info.num_cores, _info.num_subcores
_NW = _NC * _NS_SUB
_TB = 128
_NT = _BATCH // _TB
_TPW = _NT // _NW
_NU = _HIST * _TPW


def _make_kernel():
    mesh = plsc.VectorSubcoreMesh(core_axis_name="c", subcore_axis_name="s")

    @functools.partial(
        pl.kernel,
        out_type=jax.ShapeDtypeStruct(
            (_HIST, _EMBED_DIM // 8, _NT, 8, 128), jnp.float32
        ),
        mesh=mesh,
        scratch_types=(
            [pltpu.VMEM((_TPW * _TB, _HIST), jnp.int32)]
            + [pltpu.VMEM((_TB,), jnp.int32) for _ in range(2)]
            + [pltpu.VMEM((_TB, _EMBED_DIM), jnp.float32) for _ in range(2)]
            + [pltpu.VMEM((_EMBED_DIM // 8, 8, 128), jnp.float32) for _ in range(2)]
            + [pltpu.SemaphoreType.DMA for _ in range(4)]
        ),
        compiler_params=pltpu.CompilerParams(
            use_tc_tiling_on_sc=False, needs_layout_passes=False
        ),
    )
    def gather_kernel(table_hbm, idx_hbm, out_hbm, *scratch):
        idx_all = scratch[0]
        il = scratch[1:3]
        rows_in = scratch[3:5]
        rows_t = scratch[5:7]
        gsems = scratch[7:9]
        osems = scratch[9:11]
        wid = lax.axis_index("s") * _NC + lax.axis_index("c")
        iota16 = lax.iota(jnp.int32, 16)

        def build_il(s, u):
            h = u // _TPW
            base = (u % _TPW) * _TB
            hvec = jnp.full((16,), h, jnp.int32)
            for j in range(8):
                v = plsc.load_gather(idx_all, [base + j * 16 + iota16, hvec])
                il[s][pl.ds(j * 16, 16)] = v

        def gather(s):
            return pltpu.make_async_copy(table_hbm.at[il[s]], rows_in[s], gsems[s])

        def transpose(s):
            for r in range(_EMBED_DIM):
                rvec = jnp.full((16,), r, jnp.int32)
                for j in range(8):
                    v = plsc.load_gather(rows_in[s], [j * 16 + iota16, rvec])
                    rows_t[s][r // 8, r % 8, pl.ds(j * 16, 16)] = v

        def out_copies(s, u):
            h = u // _TPW
            t = wid * _TPW + u % _TPW
            return [
                pltpu.make_async_copy(rows_t[s].at[a], out_hbm.at[h, a, t], osems[s])
                for a in range(_EMBED_DIM // 8)
            ]

        def start4(s, u):
            for c in out_copies(s, u):
                c.start()

        def wait4(s, u):
            for c in out_copies(s, u):
                c.wait()

        pltpu.sync_copy(idx_hbm.at[pl.ds(wid * _TPW * _TB, _TPW * _TB)], idx_all)

        build_il(0, 0)
        gather(0).start()
        build_il(1, 1)
        gather(1).start()
        gather(0).wait()
        transpose(0)
        start4(0, 0)
        gather(1).wait()
        transpose(1)
        start4(1, 1)

        def pair_body(k, carry):
            u0 = 2 * k
            u1 = 2 * k + 1
            wait4(0, u0 - 2)
            build_il(0, u0)
            gather(0).start()
            wait4(1, u1 - 2)
            build_il(1, u1)
            gather(1).start()
            gather(0).wait()
            transpose(0)
            start4(0, u0)
            gather(1).wait()
            transpose(1)
            start4(1, u1)
            return carry

        lax.fori_loop(1, _NU // 2, pair_body, 0)
        wait4(0, _NU - 2)
        wait4(1, _NU - 1)

    return gather_kernel


_gather = _make_kernel()


def kernel(indices, weight):
    out5 = _gather(weight, indices.astype(jnp.int32))
    return out5.transpose(2, 4, 0, 1, 3).reshape(_BATCH, _HIST, _EMBED_DIM)


# speedup vs baseline: 1.0619x; 1.0619x over previous
import functools

import jax
import jax.numpy as jnp
from jax import lax
from jax.experimental import pallas as pl
from jax.experimental.pallas import tpu as pltpu
from jax.experimental.pallas import tpu_sc as plsc

_VOCAB = 1000000
_EMBED_DIM = 32
_BATCH = 16384
_HIST = 50

_info = plsc.get_sparse_core_info()
_NC, _NS_SUB = _info.num_cores, _info.num_subcores
_NW = _NC * _NS_SUB
_TB = 128
_NT = _BATCH // _TB
_TPW = _NT // _NW
_NU = _HIST * _TPW
_NSLOT = 4
assert _NU % _NSLOT == 0


def _make_kernel():
    mesh = plsc.VectorSubcoreMesh(core_axis_name="c", subcore_axis_name="s")

    @functools.partial(
        pl.kernel,
        out_type=jax.ShapeDtypeStruct(
            (_HIST, _EMBED_DIM // 8, _NT, 8, 128), jnp.float32
        ),
        mesh=mesh,
        scratch_types=(
            [pltpu.VMEM((_TPW * _TB, _HIST), jnp.int32)]
            + [pltpu.VMEM((_TB,), jnp.int32) for _ in range(_NSLOT)]
            + [pltpu.VMEM((_TB, _EMBED_DIM), jnp.float32) for _ in range(_NSLOT)]
            + [pltpu.VMEM((_EMBED_DIM // 8, 8, 128), jnp.float32) for _ in range(_NSLOT)]
            + [pltpu.SemaphoreType.DMA for _ in range(2 * _NSLOT)]
        ),
        compiler_params=pltpu.CompilerParams(
            use_tc_tiling_on_sc=False, needs_layout_passes=False
        ),
    )
    def gather_kernel(table_hbm, idx_hbm, out_hbm, *scratch):
        idx_all = scratch[0]
        il = scratch[1 : 1 + _NSLOT]
        rows_in = scratch[1 + _NSLOT : 1 + 2 * _NSLOT]
        rows_t = scratch[1 + 2 * _NSLOT : 1 + 3 * _NSLOT]
        gsems = scratch[1 + 3 * _NSLOT : 1 + 4 * _NSLOT]
        osems = scratch[1 + 4 * _NSLOT : 1 + 5 * _NSLOT]
        wid = lax.axis_index("s") * _NC + lax.axis_index("c")
        iota16 = lax.iota(jnp.int32, 16)

        def build_il(s, u):
            h = u // _TPW
            base = (u % _TPW) * _TB
            hvec = jnp.full((16,), h, jnp.int32)
            for j in range(8):
                v = plsc.load_gather(idx_all, [base + j * 16 + iota16, hvec])
                il[s][pl.ds(j * 16, 16)] = v

        def gather(s):
            return pltpu.make_async_copy(table_hbm.at[il[s]], rows_in[s], gsems[s])

        def transpose(s):
            def tr(r, carry):
                rvec = jnp.full((16,), r, jnp.int32)
                for j in range(8):
                    v = plsc.load_gather(rows_in[s], [j * 16 + iota16, rvec])
                    rows_t[s][r // 8, r % 8, pl.ds(j * 16, 16)] = v
                return carry

            lax.fori_loop(0, _EMBED_DIM, tr, 0)

        def out_copies(s, u):
            h = u // _TPW
            t = wid * _TPW + u % _TPW
            return [
                pltpu.make_async_copy(rows_t[s].at[a], out_hbm.at[h, a, t], osems[s])
                for a in range(_EMBED_DIM // 8)
            ]

        def start4(s, u):
            for c in out_copies(s, u):
                c.start()

        def wait4(s, u):
            for c in out_copies(s, u):
                c.wait()

        pltpu.sync_copy(idx_hbm.at[pl.ds(wid * _TPW * _TB, _TPW * _TB)], idx_all)

        for s in range(_NSLOT):
            build_il(s, s)
            gather(s).start()
        for s in range(_NSLOT):
            gather(s).wait()
            transpose(s)
            start4(s, s)

        def round_body(k, carry):
            for s in range(_NSLOT):
                u = _NSLOT * k + s
                wait4(s, u - _NSLOT)
                build_il(s, u)
                gather(s).start()
            for s in range(_NSLOT):
                u = _NSLOT * k + s
                gather(s).wait()
                transpose(s)
                start4(s, u)
            return carry

        lax.fori_loop(1, _NU // _NSLOT, round_body, 0)
        for s in range(_NSLOT):
            wait4(s, _NU - _NSLOT + s)

    return gather_kernel


_gather = _make_kernel()


def kernel(indices, weight):
    out5 = _gather(weight, indices.astype(jnp.int32))
    return out5.transpose(2, 4, 0, 1, 3).reshape(_BATCH, _HIST, _EMBED_DIM)


# speedup vs baseline: 1.2222x; 1.1509x over previous
import functools

import jax
import jax.numpy as jnp
from jax import lax
from jax.experimental import pallas as pl
from jax.experimental.pallas import tpu as pltpu
from jax.experimental.pallas import tpu_sc as plsc

_VOCAB = 1000000
_EMBED_DIM = 32
_BATCH = 16384
_HIST = 50

_info = plsc.get_sparse_core_info()
_NC, _NS_SUB = _info.num_cores, _info.num_subcores
_NW = _NC * _NS_SUB
_BPW = _BATCH // _NW
_NBS = 32
_NSLOT = 2
_N_ROUNDS = _BPW // (_NBS * _NSLOT)
assert _NBS * _NSLOT * _N_ROUNDS == _BPW


def _make_kernel():
    mesh = plsc.VectorSubcoreMesh(core_axis_name="c", subcore_axis_name="s")

    @functools.partial(
        pl.kernel,
        out_type=jax.ShapeDtypeStruct((_BATCH, _HIST, _EMBED_DIM), jnp.float32),
        mesh=mesh,
        scratch_types=(
            [pltpu.VMEM((_NBS, _HIST), jnp.int32) for _ in range(_NSLOT)]
            + [pltpu.VMEM((_NBS, _HIST, _EMBED_DIM), jnp.float32) for _ in range(_NSLOT)]
            + [pltpu.SemaphoreType.DMA for _ in range(3 * _NSLOT)]
        ),
        compiler_params=pltpu.CompilerParams(use_tc_tiling_on_sc=False),
    )
    def gather_kernel(table_hbm, idx_hbm, out_hbm, *scratch):
        idx_bufs = scratch[:_NSLOT]
        row_bufs = scratch[_NSLOT : 2 * _NSLOT]
        isems = scratch[2 * _NSLOT : 3 * _NSLOT]
        gsems = scratch[3 * _NSLOT : 4 * _NSLOT]
        osems = scratch[4 * _NSLOT : 5 * _NSLOT]
        wid = lax.axis_index("s") * _NC + lax.axis_index("c")
        w_base = wid * _BPW

        def idx_copy(g, s):
            off = jnp.minimum(w_base + g * _NBS, _BATCH - _NBS)
            return pltpu.make_async_copy(
                idx_hbm.at[pl.ds(off, _NBS)], idx_bufs[s], isems[s]
            )

        def gathers(s):
            return [
                pltpu.make_async_copy(
                    table_hbm.at[idx_bufs[s].at[b]], row_bufs[s].at[b], gsems[s]
                )
                for b in range(_NBS)
            ]

        def out_copy(g, s):
            return pltpu.make_async_copy(
                row_bufs[s],
                out_hbm.at[pl.ds(w_base + g * _NBS, _NBS)],
                osems[s],
            )

        for s in range(_NSLOT):
            idx_copy(s, s).start()
        for s in range(_NSLOT):
            idx_copy(s, s).wait()
            for gth in gathers(s):
                gth.start()
        for s in range(_NSLOT):
            for gth in gathers(s):
                gth.wait()
            out_copy(s, s).start()
            idx_copy(_NSLOT + s, s).start()

        def round_body(r, carry):
            for s in range(_NSLOT):
                g = r * _NSLOT + s
                out_copy(g - _NSLOT, s).wait()
                idx_copy(g, s).wait()
                for gth in gathers(s):
                    gth.start()
            for s in range(_NSLOT):
                g = r * _NSLOT + s
                for gth in gathers(s):
                    gth.wait()
                out_copy(g, s).start()
                idx_copy(g + _NSLOT, s).start()
            return carry

        lax.fori_loop(1, _N_ROUNDS, round_body, 0)

        for s in range(_NSLOT):
            out_copy((_N_ROUNDS - 1) * _NSLOT + s, s).wait()
            idx_copy(0, s).wait()

    return gather_kernel


_gather = _make_kernel()


def kernel(indices, weight):
    return _gather(weight, indices.astype(jnp.int32))
